# baseline (speedup 1.0000x reference)
import functools

import jax
import jax.numpy as jnp
from jax import lax
from jax.experimental import pallas as pl
from jax.experimental.pallas import tpu as pltpu
from jax.experimental.pallas import tpu_sc as plsc

N = 4096
D = 64
K = 16
KM1 = K - 1
NSAMP = N * KM1

_TINY = 1.1754943508222875e-38
_KS0 = 0
_KS1 = 42
_KS2 = _KS0 ^ _KS1 ^ 0x1BD11BDA
_ROTS = ((13, 15, 26, 6), (17, 29, 16, 24))


def _threefry_bits(lo_u32):
    ks = (_KS0, _KS1, _KS2)
    x0 = jnp.zeros_like(lo_u32)
    x1 = lo_u32 + jnp.uint32(ks[1])
    for g in range(5):
        for r in _ROTS[g % 2]:
            x0 = x0 + x1
            x1 = (x1 << jnp.uint32(r)) | (x1 >> jnp.uint32(32 - r))
            x1 = x1 ^ x0
        x0 = x0 + jnp.uint32(ks[(g + 1) % 3])
        x1 = x1 + jnp.uint32((ks[(g + 2) % 3] + g + 1) & 0xFFFFFFFF)
    return x0 ^ x1


def _log_weights(sim, row0, rows, cols):
    gi = lax.broadcasted_iota(jnp.int32, (rows, cols), 0) + row0
    jj = lax.broadcasted_iota(jnp.int32, (rows, cols), 1)
    dist = 2.0 - 2.0 * sim
    dist = dist + jnp.where(gi == jj, jnp.float32(1.0), jnp.float32(0.0))
    dist = jnp.sqrt(jnp.maximum(dist, 0.0))
    dist = jnp.maximum(dist, jnp.float32(0.5))
    lw = (-62.0) * jnp.log(dist) - 30.5 * jnp.log(
        jnp.maximum(1.0 - 0.25 * (dist * dist), jnp.float32(1e-8)))
    return gi, jj, dist, lw


def _gmax_body(x_ref, xt_ref, o_ref, a_ref, p_ref):
    i = pl.program_id(0)
    row0 = i * 512
    sim = jnp.dot(x_ref[...], xt_ref[...], preferred_element_type=jnp.float32)
    _, _, _, lw = _log_weights(sim, row0, 512, N)
    m = jnp.max(lw)

    @pl.when(i == 0)
    def _():
        o_ref[0, 0] = m

    @pl.when(i > 0)
    def _():
        o_ref[0, 0] = jnp.maximum(o_ref[0, 0], m)

    ri = lax.broadcasted_iota(jnp.int32, (512, K), 0) + row0
    si = lax.broadcasted_iota(jnp.int32, (512, K), 1)
    a_ref[...] = ri
    p_ref[...] = (ri // K) * K + si + (si >= (ri % K)).astype(jnp.int32)


def _gmax_call(x, xt, interpret=False):
    return pl.pallas_call(
        _gmax_body,
        grid=(8,),
        in_specs=[
            pl.BlockSpec((512, D), lambda i: (i, 0)),
            pl.BlockSpec((D, N), lambda i: (0, 0)),
        ],
        out_specs=[
            pl.BlockSpec(memory_space=pltpu.SMEM),
            pl.BlockSpec((512, K), lambda i: (i, 0)),
            pl.BlockSpec((512, K), lambda i: (i, 0)),
        ],
        out_shape=[
            jax.ShapeDtypeStruct((1, 1), jnp.float32),
            jax.ShapeDtypeStruct((N, K), jnp.int32),
            jax.ShapeDtypeStruct((N, K), jnp.int32),
        ],
        interpret=interpret,
    )(x, xt)


ROWS_B = 64
CHUNK = 512
SGROUP = 15
RSC = 960
N_TC = N - RSC


def _sample_body(x_ref, xt_ref, gmax_ref, nidx_ref, q_ref):
    step = pl.program_id(0)
    row0 = step * ROWS_B
    gmax = gmax_ref[0, 0]

    sim = jnp.dot(x_ref[...], xt_ref[...], preferred_element_type=jnp.float32)
    gi, jj, dist, lw = _log_weights(sim, row0, ROWS_B, N)
    sel = ((gi // K) != (jj // K)) & (dist < jnp.float32(1.4))
    w = jnp.where(sel, jnp.exp(lw - gmax), jnp.float32(0.0)) + jnp.float32(1e-8)
    q_ref[...] = 1.0 / w

    def row_group(rg, _):
        grow0 = row0 + rg * 8

        def one_sample_group(t, acc):
            iot0 = lax.broadcasted_iota(jnp.int32, (8, CHUNK), 0)
            iot1 = lax.broadcasted_iota(jnp.int32, (8, CHUNK), 1)
            rowpart = (grow0 + iot0) * N + iot1
            jf_base = iot1.astype(jnp.float32)
            lane16 = lax.broadcasted_iota(jnp.int32, (8, K), 1)
            for ds in range(SGROUP):
                s = t * SGROUP + ds
                bz = jnp.full((8, 1), 3.4e38, jnp.float32)
                bj = jnp.zeros((8, 1), jnp.float32)
                for c in range(N // CHUNK):
                    lo = (rowpart
                          + (s * 16777216 + c * CHUNK)).astype(jnp.uint32)
                    bits = _threefry_bits(lo)
                    fb = lax.bitcast_convert_type(
                        (bits >> jnp.uint32(9)) | jnp.uint32(0x3F800000),
                        jnp.float32)
                    u = jnp.maximum(fb - 1.0, jnp.float32(_TINY))
                    e = -jnp.log(u)
                    z = e * q_ref[pl.ds(rg * 8, 8), pl.ds(c * CHUNK, CHUNK)]
                    cmin = jnp.min(z, axis=1, keepdims=True)
                    cidx = jnp.min(
                        jnp.where(z == cmin, jf_base + jnp.float32(c * CHUNK),
                                  jnp.float32(4e9)),
                        axis=1, keepdims=True)
                    upd = cmin < bz
                    bj = jnp.where(upd, cidx, bj)
                    bz = jnp.minimum(bz, cmin)
                acc = jnp.where(lane16 == s, bj.astype(jnp.int32), acc)
            return acc

        acc = lax.fori_loop(0, KM1 // SGROUP, one_sample_group,
                            jnp.zeros((8, K), jnp.int32))
        nidx_ref[pl.ds(rg * 8, 8), :] = acc
        return 0

    lax.fori_loop(0, ROWS_B // 8, row_group, 0)


def _sample_call(x, xt, gmax, interpret=False):
    return pl.pallas_call(
        _sample_body,
        grid=(N_TC // ROWS_B,),
        in_specs=[
            pl.BlockSpec((ROWS_B, D), lambda i: (i, 0)),
            pl.BlockSpec((D, N), lambda i: (0, 0)),
            pl.BlockSpec(memory_space=pltpu.SMEM),
        ],
        out_specs=pl.BlockSpec((ROWS_B, K), lambda i: (i, 0)),
        out_shape=jax.ShapeDtypeStruct((N_TC, K), jnp.int32),
        scratch_shapes=[pltpu.VMEM((ROWS_B, N), jnp.float32)],
        interpret=interpret,
    )(x, xt, gmax)


def _qsc_body(x_ref, xt_ref, gmax_ref, q_ref):
    i = pl.program_id(0)
    row0 = N_TC + i * ROWS_B
    gmax = gmax_ref[0, 0]
    sim = jnp.dot(x_ref[...], xt_ref[...], preferred_element_type=jnp.float32)
    gi, jj, dist, lw = _log_weights(sim, row0, ROWS_B, N)
    sel = ((gi // K) != (jj // K)) & (dist < jnp.float32(1.4))
    w = jnp.where(sel, jnp.exp(lw - gmax), jnp.float32(0.0)) + jnp.float32(1e-8)
    q_ref[...] = 1.0 / w


def _qsc_call(x, xt, gmax, interpret=False):
    return pl.pallas_call(
        _qsc_body,
        grid=(RSC // ROWS_B,),
        in_specs=[
            pl.BlockSpec((ROWS_B, D), lambda i: (N_TC // ROWS_B + i, 0)),
            pl.BlockSpec((D, N), lambda i: (0, 0)),
            pl.BlockSpec(memory_space=pltpu.SMEM),
        ],
        out_specs=pl.BlockSpec((ROWS_B, N), lambda i: (i, 0)),
        out_shape=jax.ShapeDtypeStruct((RSC, N), jnp.float32),
        interpret=interpret,
    )(x, xt, gmax)


RPT = RSC // 32
SC_UNROLL = 16


def _sc_log(u):
    b = lax.bitcast_convert_type(u, jnp.int32)
    expo = (b >> 23) - 127
    m = lax.bitcast_convert_type((b & 0x7FFFFF) | 0x3F800000, jnp.float32)
    big = m > jnp.float32(1.4142135623730951)
    m = jnp.where(big, m * 0.5, m)
    ef = (expo + big.astype(jnp.int32)).astype(jnp.float32)
    t = m - 1.0
    s = t / (2.0 + t)
    p = s * s
    poly = 2.0 * s * (1.0 + p * (0.33333333333333333 + p * (
        0.2 + p * (0.14285714285714285 + p * 0.1111111111111111))))
    return ef * jnp.float32(0.6931471805599453) + poly


def _sc_sample_call(q_sc):
    mesh = plsc.VectorSubcoreMesh(core_axis_name="c", subcore_axis_name="s")

    @functools.partial(
        pl.kernel,
        mesh=mesh,
        out_type=jax.ShapeDtypeStruct((32, RPT, K), jnp.int32),
        scratch_types=[
            pltpu.VMEM((N,), jnp.float32),
            pltpu.VMEM((RPT, K), jnp.int32),
            pltpu.VMEM((K,), jnp.float32),
            pltpu.VMEM((K,), jnp.int32),
            pltpu.SemaphoreType.DMA,
        ],
        compiler_params=pltpu.CompilerParams(needs_layout_passes=False),
    )
    def body(q_hbm, out_hbm, qbuf, nbuf, bzbuf, bjbuf, sem):
        wid = lax.axis_index("s") * 2 + lax.axis_index("c")
        lane = lax.broadcasted_iota(jnp.int32, (K,), 0)

        def one_row(r, _):
            grow = wid * RPT + r
            pltpu.sync_copy(q_hbm.at[grow], qbuf)
            rowbase = (N_TC + grow) * N

            def one_s(s, _):
                sbase = rowbase + s * 16777216
                bzbuf[...] = jnp.full((K,), 3.4e38, jnp.float32)
                bjbuf[...] = jnp.zeros((K,), jnp.int32)

                def jv_group(g, _):
                    bz = bzbuf[...]
                    bj = bjbuf[...]
                    for u in range(SC_UNROLL):
                        j0 = (g * SC_UNROLL + u) * K
                        lo = (sbase + j0 + lane).astype(jnp.uint32)
                        bits = _threefry_bits(lo)
                        fb = lax.bitcast_convert_type(
                            (bits >> jnp.uint32(9)) | jnp.uint32(0x3F800000),
                            jnp.float32)
                        uu = jnp.maximum(fb - 1.0, jnp.float32(_TINY))
                        e = -_sc_log(uu)
                        z = e * qbuf[pl.ds(j0, K)]
                        m = z < bz
                        bz = jnp.where(m, z, bz)
                        bj = jnp.where(m, j0 + lane, bj)
                    bzbuf[...] = bz
                    bjbuf[...] = bj
                    return 0

                lax.fori_loop(0, N // K // SC_UNROLL, jv_group, 0)
                bz = bzbuf[...]
                bj = bjbuf[...]
                zmin = jnp.broadcast_to(jnp.min(bz, axis=0), (K,))
                cand = jnp.where(bz == zmin, bj, jnp.int32(2 * N))
                idx = jnp.broadcast_to(jnp.min(cand, axis=0), (K,))
                sv = jnp.broadcast_to(s, (K,))
                nbuf[r, :] = jnp.where(lane == sv, idx, nbuf[r, :])
                return 0

            nbuf[r, :] = jnp.zeros((K,), jnp.int32)
            lax.fori_loop(0, KM1, one_s, 0)
            return 0

        lax.fori_loop(0, RPT, one_row, 0)
        pltpu.sync_copy(nbuf, out_hbm.at[wid])

    return body(q_sc).reshape(RSC, K)


B_PER_W = NSAMP // 32
IDX_ROWS_PER_W = B_PER_W // 128


GATHER_PASSES = 3
CHUNKS_PER_PASS = IDX_ROWS_PER_W // GATHER_PASSES
ROWS_PER_PASS = CHUNKS_PER_PASS * 128


def _make_gather(n_out):
    mesh = plsc.VectorSubcoreMesh(core_axis_name="c", subcore_axis_name="s")
    row_f32 = jax.ShapeDtypeStruct((NSAMP, 128), jnp.float32)

    @functools.partial(
        pl.kernel,
        mesh=mesh,
        out_type=[row_f32] * n_out,
        scratch_types=[
            pltpu.VMEM((IDX_ROWS_PER_W, 128), jnp.int32),
            pltpu.VMEM((ROWS_PER_PASS, 128), jnp.float32),
            pltpu.SemaphoreType.DMA,
        ],
    )
    def body(x_hbm, *rest):
        idx_hbms = rest[:n_out]
        outs = rest[n_out:2 * n_out]
        idx_v, rows_v, sem = rest[2 * n_out:]
        wid = lax.axis_index("s") * 2 + lax.axis_index("c")
        base = wid * B_PER_W
        for idx_hbm, out_hbm in zip(idx_hbms, outs):
            pltpu.sync_copy(idx_hbm.at[wid], idx_v)
            for p in range(GATHER_PASSES):
                cps = [
                    pltpu.async_copy(
                        x_hbm.at[idx_v.at[p * CHUNKS_PER_PASS + c]],
                        rows_v.at[pl.ds(c * 128, 128)], sem)
                    for c in range(CHUNKS_PER_PASS)
                ]
                for cp in cps:
                    cp.wait()
                pltpu.sync_copy(
                    rows_v,
                    out_hbm.at[pl.ds(base + p * ROWS_PER_PASS, ROWS_PER_PASS)])

    return body


def kernel(x):
    xt = x.T
    xp128 = jnp.pad(x, ((0, 0), (0, 128 - D)))
    gmax, a16, p16 = _gmax_call(x, xt)
    a_flat = a16[:, :KM1].reshape(-1)
    p_flat = p16[:, :KM1].reshape(-1)
    xa, xp = _make_gather(2)(
        xp128,
        a_flat.reshape(32, IDX_ROWS_PER_W, 128),
        p_flat.reshape(32, IDX_ROWS_PER_W, 128),
    )
    q_sc = _qsc_call(x, xt, gmax)
    nidx_sc = _sc_sample_call(q_sc)
    nidx_tc = _sample_call(x, xt, gmax)
    nidx16 = jnp.concatenate([nidx_tc, nidx_sc], axis=0)
    n_flat = nidx16[:, :KM1].reshape(-1)
    (xn,) = _make_gather(1)(
        xp128,
        n_flat.reshape(32, IDX_ROWS_PER_W, 128),
    )
    return (a_flat, xa[:, :D], xp[:, :D], xn[:, :D], x)


# speedup vs baseline: 1.0658x; 1.0658x over previous
import functools

import jax
import jax.numpy as jnp
from jax import lax
from jax.experimental import pallas as pl
from jax.experimental.pallas import tpu as pltpu
from jax.experimental.pallas import tpu_sc as plsc

N = 4096
D = 64
K = 16
KM1 = K - 1
NSAMP = N * KM1

_TINY = 1.1754943508222875e-38
_KS0 = 0
_KS1 = 42
_KS2 = _KS0 ^ _KS1 ^ 0x1BD11BDA
_ROTS = ((13, 15, 26, 6), (17, 29, 16, 24))


def _threefry_bits(lo_u32):
    ks = (_KS0, _KS1, _KS2)
    x0 = jnp.zeros_like(lo_u32)
    x1 = lo_u32 + jnp.uint32(ks[1])
    for g in range(5):
        for r in _ROTS[g % 2]:
            x0 = x0 + x1
            x1 = (x1 << jnp.uint32(r)) | (x1 >> jnp.uint32(32 - r))
            x1 = x1 ^ x0
        x0 = x0 + jnp.uint32(ks[(g + 1) % 3])
        x1 = x1 + jnp.uint32((ks[(g + 2) % 3] + g + 1) & 0xFFFFFFFF)
    return x0 ^ x1


def _log_weights(sim, row0, rows, cols):
    gi = lax.broadcasted_iota(jnp.int32, (rows, cols), 0) + row0
    jj = lax.broadcasted_iota(jnp.int32, (rows, cols), 1)
    dist = 2.0 - 2.0 * sim
    dist = dist + jnp.where(gi == jj, jnp.float32(1.0), jnp.float32(0.0))
    dist = jnp.sqrt(jnp.maximum(dist, 0.0))
    dist = jnp.maximum(dist, jnp.float32(0.5))
    lw = (-62.0) * jnp.log(dist) - 30.5 * jnp.log(
        jnp.maximum(1.0 - 0.25 * (dist * dist), jnp.float32(1e-8)))
    return gi, jj, dist, lw


def _gmax_body(x_ref, xt_ref, o_ref, a_ref, p_ref):
    i = pl.program_id(0)
    row0 = i * 512
    sim = jnp.dot(x_ref[...], xt_ref[...], preferred_element_type=jnp.float32)
    _, _, _, lw = _log_weights(sim, row0, 512, N)
    m = jnp.max(lw)

    @pl.when(i == 0)
    def _():
        o_ref[0, 0] = m

    @pl.when(i > 0)
    def _():
        o_ref[0, 0] = jnp.maximum(o_ref[0, 0], m)

    ri = lax.broadcasted_iota(jnp.int32, (512, K), 0) + row0
    si = lax.broadcasted_iota(jnp.int32, (512, K), 1)
    a_ref[...] = ri
    p_ref[...] = (ri // K) * K + si + (si >= (ri % K)).astype(jnp.int32)


def _gmax_call(x, xt):
    return pl.pallas_call(
        _gmax_body,
        grid=(8,),
        in_specs=[
            pl.BlockSpec((512, D), lambda i: (i, 0)),
            pl.BlockSpec((D, N), lambda i: (0, 0)),
        ],
        out_specs=[
            pl.BlockSpec(memory_space=pltpu.SMEM),
            pl.BlockSpec((512, K), lambda i: (i, 0)),
            pl.BlockSpec((512, K), lambda i: (i, 0)),
        ],
        out_shape=[
            jax.ShapeDtypeStruct((1, 1), jnp.float32),
            jax.ShapeDtypeStruct((N, K), jnp.int32),
            jax.ShapeDtypeStruct((N, K), jnp.int32),
        ],
    )(x, xt)


ROWS_B = 128
CHUNK = 512
SGROUP = 15
RSC = 896
N_TC = N - RSC


def _sample_body(x_ref, xt_ref, gmax_ref, nidx_ref, q_ref):
    step = pl.program_id(0)
    row0 = step * ROWS_B
    gmax = gmax_ref[0, 0]

    sim = jnp.dot(x_ref[...], xt_ref[...], preferred_element_type=jnp.float32)
    gi, jj, dist, lw = _log_weights(sim, row0, ROWS_B, N)
    sel = ((gi // K) != (jj // K)) & (dist < jnp.float32(1.4))
    w = jnp.where(sel, jnp.exp(lw - gmax), jnp.float32(0.0)) + jnp.float32(1e-8)
    q_ref[...] = 1.0 / w

    def row_group(rg, _):
        grow0 = row0 + rg * 8

        def one_sample_group(t, acc):
            iot0 = lax.broadcasted_iota(jnp.int32, (8, CHUNK), 0)
            iot1 = lax.broadcasted_iota(jnp.int32, (8, CHUNK), 1)
            rowpart = (grow0 + iot0) * N + iot1
            jf_base = iot1.astype(jnp.float32)
            lane16 = lax.broadcasted_iota(jnp.int32, (8, K), 1)
            for ds in range(SGROUP):
                s = t * SGROUP + ds
                bz = jnp.full((8, 1), 3.4e38, jnp.float32)
                bj = jnp.zeros((8, 1), jnp.float32)
                for c in range(N // CHUNK):
                    lo = (rowpart
                          + (s * 16777216 + c * CHUNK)).astype(jnp.uint32)
                    bits = _threefry_bits(lo)
                    fb = lax.bitcast_convert_type(
                        (bits >> jnp.uint32(9)) | jnp.uint32(0x3F800000),
                        jnp.float32)
                    u = jnp.maximum(fb - 1.0, jnp.float32(_TINY))
                    e = -jnp.log(u)
                    z = e * q_ref[pl.ds(rg * 8, 8), pl.ds(c * CHUNK, CHUNK)]
                    cmin = jnp.min(z, axis=1, keepdims=True)
                    cidx = jnp.min(
                        jnp.where(z == cmin, jf_base + jnp.float32(c * CHUNK),
                                  jnp.float32(4e9)),
                        axis=1, keepdims=True)
                    upd = cmin < bz
                    bj = jnp.where(upd, cidx, bj)
                    bz = jnp.minimum(bz, cmin)
                acc = jnp.where(lane16 == s, bj.astype(jnp.int32), acc)
            return acc

        acc = lax.fori_loop(0, KM1 // SGROUP, one_sample_group,
                            jnp.zeros((8, K), jnp.int32))
        nidx_ref[pl.ds(rg * 8, 8), :] = acc
        return 0

    lax.fori_loop(0, ROWS_B // 8, row_group, 0)


def _sample_call(x, xt, gmax):
    return pl.pallas_call(
        _sample_body,
        grid=(N_TC // ROWS_B,),
        in_specs=[
            pl.BlockSpec((ROWS_B, D), lambda i: (i, 0)),
            pl.BlockSpec((D, N), lambda i: (0, 0)),
            pl.BlockSpec(memory_space=pltpu.SMEM),
        ],
        out_specs=pl.BlockSpec((ROWS_B, K), lambda i: (i, 0)),
        out_shape=jax.ShapeDtypeStruct((N_TC, K), jnp.int32),
        scratch_shapes=[pltpu.VMEM((ROWS_B, N), jnp.float32)],
    )(x, xt, gmax)


def _qsc_body(x_ref, xt_ref, gmax_ref, q_ref):
    i = pl.program_id(0)
    row0 = N_TC + i * ROWS_B
    gmax = gmax_ref[0, 0]
    sim = jnp.dot(x_ref[...], xt_ref[...], preferred_element_type=jnp.float32)
    gi, jj, dist, lw = _log_weights(sim, row0, ROWS_B, N)
    sel = ((gi // K) != (jj // K)) & (dist < jnp.float32(1.4))
    w = jnp.where(sel, jnp.exp(lw - gmax), jnp.float32(0.0)) + jnp.float32(1e-8)
    q_ref[...] = 1.0 / w


def _qsc_call(x, xt, gmax):
    return pl.pallas_call(
        _qsc_body,
        grid=(RSC // ROWS_B,),
        in_specs=[
            pl.BlockSpec((ROWS_B, D), lambda i: (N_TC // ROWS_B + i, 0)),
            pl.BlockSpec((D, N), lambda i: (0, 0)),
            pl.BlockSpec(memory_space=pltpu.SMEM),
        ],
        out_specs=pl.BlockSpec((ROWS_B, N), lambda i: (i, 0)),
        out_shape=jax.ShapeDtypeStruct((RSC, N), jnp.float32),
    )(x, xt, gmax)


RPT = RSC // 32
SC_UNROLL = 16


def _sc_log(u):
    b = lax.bitcast_convert_type(u, jnp.int32)
    expo = (b >> 23) - 127
    m = lax.bitcast_convert_type((b & 0x7FFFFF) | 0x3F800000, jnp.float32)
    big = m > jnp.float32(1.4142135623730951)
    m = jnp.where(big, m * 0.5, m)
    ef = (expo + big.astype(jnp.int32)).astype(jnp.float32)
    t = m - 1.0
    s = t / (2.0 + t)
    p = s * s
    poly = 2.0 * s * (1.0 + p * (0.33333333333333333 + p * (
        0.2 + p * (0.14285714285714285 + p * 0.1111111111111111))))
    return ef * jnp.float32(0.6931471805599453) + poly


def _sc_sample_call(q_sc):
    mesh = plsc.VectorSubcoreMesh(core_axis_name="c", subcore_axis_name="s")

    @functools.partial(
        pl.kernel,
        mesh=mesh,
        out_type=jax.ShapeDtypeStruct((32, RPT, K), jnp.int32),
        scratch_types=[
            pltpu.VMEM((N,), jnp.float32),
            pltpu.VMEM((RPT, K), jnp.int32),
            pltpu.VMEM((K,), jnp.float32),
            pltpu.VMEM((K,), jnp.int32),
            pltpu.SemaphoreType.DMA,
        ],
        compiler_params=pltpu.CompilerParams(needs_layout_passes=False),
    )
    def body(q_hbm, out_hbm, qbuf, nbuf, bzbuf, bjbuf, sem):
        wid = lax.axis_index("s") * 2 + lax.axis_index("c")
        lane = lax.broadcasted_iota(jnp.int32, (K,), 0)

        def one_row(r, _):
            grow = wid * RPT + r
            pltpu.sync_copy(q_hbm.at[grow], qbuf)
            rowbase = (N_TC + grow) * N

            def one_s(s, _):
                sbase = rowbase + s * 16777216
                bzbuf[...] = jnp.full((K,), 3.4e38, jnp.float32)
                bjbuf[...] = jnp.zeros((K,), jnp.int32)

                def jv_group(g, _):
                    bz = bzbuf[...]
                    bj = bjbuf[...]
                    for u in range(SC_UNROLL):
                        j0 = (g * SC_UNROLL + u) * K
                        lo = (sbase + j0 + lane).astype(jnp.uint32)
                        bits = _threefry_bits(lo)
                        fb = lax.bitcast_convert_type(
                            (bits >> jnp.uint32(9)) | jnp.uint32(0x3F800000),
                            jnp.float32)
                        uu = jnp.maximum(fb - 1.0, jnp.float32(_TINY))
                        e = -_sc_log(uu)
                        z = e * qbuf[pl.ds(j0, K)]
                        m = z < bz
                        bz = jnp.where(m, z, bz)
                        bj = jnp.where(m, j0 + lane, bj)
                    bzbuf[...] = bz
                    bjbuf[...] = bj
                    return 0

                lax.fori_loop(0, N // K // SC_UNROLL, jv_group, 0)
                bz = bzbuf[...]
                bj = bjbuf[...]
                zmin = jnp.broadcast_to(jnp.min(bz, axis=0), (K,))
                cand = jnp.where(bz == zmin, bj, jnp.int32(2 * N))
                idx = jnp.broadcast_to(jnp.min(cand, axis=0), (K,))
                sv = jnp.broadcast_to(s, (K,))
                nbuf[r, :] = jnp.where(lane == sv, idx, nbuf[r, :])
                return 0

            nbuf[r, :] = jnp.zeros((K,), jnp.int32)
            lax.fori_loop(0, KM1, one_s, 0)
            return 0

        lax.fori_loop(0, RPT, one_row, 0)
        pltpu.sync_copy(nbuf, out_hbm.at[wid])

    return body(q_sc).reshape(RSC, K)


B_PER_W = NSAMP // 32
IDX_ROWS_PER_W = B_PER_W // 128


GATHER_PASSES = 3
CHUNKS_PER_PASS = IDX_ROWS_PER_W // GATHER_PASSES
ROWS_PER_PASS = CHUNKS_PER_PASS * 128


def _make_gather(n_out):
    mesh = plsc.VectorSubcoreMesh(core_axis_name="c", subcore_axis_name="s")
    row_f32 = jax.ShapeDtypeStruct((NSAMP, 128), jnp.float32)

    @functools.partial(
        pl.kernel,
        mesh=mesh,
        out_type=[row_f32] * n_out,
        scratch_types=[
            pltpu.VMEM((IDX_ROWS_PER_W, 128), jnp.int32),
            pltpu.VMEM((ROWS_PER_PASS, 128), jnp.float32),
            pltpu.SemaphoreType.DMA,
        ],
    )
    def body(x_hbm, *rest):
        idx_hbms = rest[:n_out]
        outs = rest[n_out:2 * n_out]
        idx_v, rows_v, sem = rest[2 * n_out:]
        wid = lax.axis_index("s") * 2 + lax.axis_index("c")
        base = wid * B_PER_W
        for idx_hbm, out_hbm in zip(idx_hbms, outs):
            pltpu.sync_copy(idx_hbm.at[wid], idx_v)
            for p in range(GATHER_PASSES):
                cps = [
                    pltpu.async_copy(
                        x_hbm.at[idx_v.at[p * CHUNKS_PER_PASS + c]],
                        rows_v.at[pl.ds(c * 128, 128)], sem)
                    for c in range(CHUNKS_PER_PASS)
                ]
                for cp in cps:
                    cp.wait()
                pltpu.sync_copy(
                    rows_v,
                    out_hbm.at[pl.ds(base + p * ROWS_PER_PASS, ROWS_PER_PASS)])

    return body


def kernel(x):
    xt = x.T
    xp128 = jnp.pad(x, ((0, 0), (0, 128 - D)))
    gmax, a16, p16 = _gmax_call(x, xt)
    a_flat = a16[:, :KM1].reshape(-1)
    p_flat = p16[:, :KM1].reshape(-1)
    xa, xp = _make_gather(2)(
        xp128,
        a_flat.reshape(32, IDX_ROWS_PER_W, 128),
        p_flat.reshape(32, IDX_ROWS_PER_W, 128),
    )
    q_sc = _qsc_call(x, xt, gmax)
    nidx_sc = _sc_sample_call(q_sc)
    nidx_tc = _sample_call(x, xt, gmax)
    nidx16 = jnp.concatenate([nidx_tc, nidx_sc], axis=0)
    n_flat = nidx16[:, :KM1].reshape(-1)
    (xn,) = _make_gather(1)(
        xp128,
        n_flat.reshape(32, IDX_ROWS_PER_W, 128),
    )
    return (a_flat, xa[:, :D], xp[:, :D], xn[:, :D], x)
